# baseline (speedup 1.0000x reference)
import functools

import jax
import jax.numpy as jnp
from jax import lax
from jax.experimental import pallas as pl
from jax.experimental.pallas import tpu as pltpu
from jax.experimental.pallas import tpu_sc as plsc

_VOCAB = 1000000
_D = 32
_B = 4096
_L = 200
_NC = 2
_NS = 16
_NW = _NC * _NS
_SPW = _B // _NW
_BLKV = 32768
_VPAD = 1000448
_GRID = (_VPAD + _BLKV - 1) // _BLKV


def _mv_body(tT_ref, w_ref, p_ref):
    p_ref[...] = jnp.sum(tT_ref[...] * w_ref[...], axis=0)


def _matvec(tT, w2):
    return pl.pallas_call(
        _mv_body,
        grid=(_GRID,),
        in_specs=[
            pl.BlockSpec((_D, _BLKV), lambda i: (0, i)),
            pl.BlockSpec((_D, 1), lambda i: (0, 0)),
        ],
        out_specs=pl.BlockSpec((_BLKV,), lambda i: (i,)),
        out_shape=jax.ShapeDtypeStruct((_VPAD,), jnp.float32),
        compiler_params=pltpu.CompilerParams(
            dimension_semantics=("arbitrary",),
        ),
    )(tT, w2)


_PSLICE = _VPAD // 8


def _sc_body(tokT_hbm, p_hbm, b_hbm, out_hbm,
             p_sh, idx_v, vals_v, out_v, b_v, sem):
    cid = lax.axis_index("c")
    sid = lax.axis_index("s")
    wid = sid * _NC + cid

    @pl.when(sid < 8)
    def _stage_p():
        off = sid * _PSLICE
        pltpu.sync_copy(p_hbm.at[pl.ds(off, _PSLICE)],
                        p_sh.at[pl.ds(off, _PSLICE)])

    pltpu.sync_copy(b_hbm, b_v)
    col = wid * _SPW
    pltpu.sync_copy(tokT_hbm.at[:, pl.ds(col, _SPW)], idx_v)
    plsc.subcore_barrier()

    def fire(t, c):
        pltpu.async_copy(p_sh.at[idx_v.at[t]], vals_v.at[t], sem)
        return c

    lax.fori_loop(0, _L, fire, 0)

    def drain(t, c):
        pltpu.make_async_copy(p_sh.at[idx_v.at[0]], vals_v.at[0], sem).wait()
        return c

    lax.fori_loop(0, _L, drain, 0)

    bias = b_v[pl.ds(0, 16)][0]

    def tok_body(t, acc):
        return tuple(
            acc[g] + vals_v[t, pl.ds(16 * g, 16)] for g in range(_SPW // 16)
        )

    z = jnp.zeros((16,), jnp.float32)
    acc = lax.fori_loop(0, _L, tok_body, (z,) * (_SPW // 16))
    for g in range(_SPW // 16):
        out_v[pl.ds(16 * g, 16)] = acc[g] + bias
    pltpu.sync_copy(out_v, out_hbm.at[pl.ds(col, _SPW)])


def _sc_gather(tokT, p, b16):
    mesh = plsc.VectorSubcoreMesh(core_axis_name="c", subcore_axis_name="s")
    run = functools.partial(
        pl.kernel,
        mesh=mesh,
        out_type=jax.ShapeDtypeStruct((_B,), jnp.float32),
        scratch_types=[
            pltpu.VMEM_SHARED((_VPAD,), jnp.float32),
            pltpu.VMEM((_L, _SPW), jnp.int32),
            pltpu.VMEM((_L, _SPW), jnp.float32),
            pltpu.VMEM((_SPW,), jnp.float32),
            pltpu.VMEM((128,), jnp.float32),
            pltpu.SemaphoreType.DMA,
        ],
        compiler_params=pltpu.CompilerParams(
            needs_layout_passes=False, use_tc_tiling_on_sc=True),
    )(_sc_body)
    return run(tokT, p, b16)


def kernel(token_ids, embedding_table, W, b):
    tT = embedding_table.T
    tokT = token_ids.T
    w2 = (W * (1.0 / _L)).astype(jnp.float32)
    b16 = jnp.broadcast_to(b, (128,)).astype(jnp.float32)
    p = _matvec(tT, w2)
    out = _sc_gather(tokT, p, b16)
    return out.reshape(_B, 1)


# speedup vs baseline: 1.1140x; 1.1140x over previous
import functools

import jax
import jax.numpy as jnp
from jax import lax
from jax.experimental import pallas as pl
from jax.experimental.pallas import tpu as pltpu
from jax.experimental.pallas import tpu_sc as plsc

_VOCAB = 1000000
_D = 32
_B = 4096
_L = 200
_NC = 2
_NS = 16
_NW = _NC * _NS
_SPW = _B // _NW
_BLKV = 65536
_VPAD = 1001472
_GRID = (_VPAD + _BLKV - 1) // _BLKV


def _mv_body(tT_ref, w_ref, p_ref):
    p_ref[...] = jnp.sum(tT_ref[...] * w_ref[...], axis=0)


def _matvec(tT, w2):
    return pl.pallas_call(
        _mv_body,
        grid=(_GRID,),
        in_specs=[
            pl.BlockSpec((_D, _BLKV), lambda i: (0, i)),
            pl.BlockSpec((_D, 1), lambda i: (0, 0)),
        ],
        out_specs=pl.BlockSpec((_BLKV,), lambda i: (i,)),
        out_shape=jax.ShapeDtypeStruct((_VPAD,), jnp.float32),
        compiler_params=pltpu.CompilerParams(
            dimension_semantics=("arbitrary",),
        ),
    )(tT, w2)


_PSLICE = _VPAD // 16


def _sc_body(tokT_hbm, p_hbm, b_hbm, out_hbm,
             p_sh, idx_v, vals_v, out_v, b_v, sem, psem):
    cid = lax.axis_index("c")
    sid = lax.axis_index("s")
    wid = sid * _NC + cid

    off = sid * _PSLICE
    pcopy = pltpu.async_copy(p_hbm.at[pl.ds(off, _PSLICE)],
                             p_sh.at[pl.ds(off, _PSLICE)], psem)
    pltpu.sync_copy(b_hbm, b_v)
    col = wid * _SPW
    pltpu.sync_copy(tokT_hbm.at[:, pl.ds(col, _SPW)], idx_v)
    pcopy.wait()
    plsc.subcore_barrier()

    def fire(t, c):
        pltpu.async_copy(p_sh.at[idx_v.at[t]], vals_v.at[t], sem)
        return c

    lax.fori_loop(0, _L, fire, 0)

    def drain(t, c):
        pltpu.make_async_copy(p_sh.at[idx_v.at[0]], vals_v.at[0], sem).wait()
        return c

    lax.fori_loop(0, _L, drain, 0)

    bias = b_v[pl.ds(0, 16)][0]

    def tok_body(t, acc):
        return tuple(
            acc[g] + vals_v[t, pl.ds(16 * g, 16)]
            for g in range(_SPW // 16)
        )

    z = jnp.zeros((16,), jnp.float32)
    acc = lax.fori_loop(0, _L, tok_body, (z,) * (_SPW // 16))
    for g in range(_SPW // 16):
        out_v[pl.ds(16 * g, 16)] = acc[g] + bias
    pltpu.sync_copy(out_v, out_hbm.at[pl.ds(col, _SPW)])


def _sc_gather(tokT, p, b16):
    mesh = plsc.VectorSubcoreMesh(core_axis_name="c", subcore_axis_name="s")
    run = functools.partial(
        pl.kernel,
        mesh=mesh,
        out_type=jax.ShapeDtypeStruct((_B,), jnp.float32),
        scratch_types=[
            pltpu.VMEM_SHARED((_VPAD,), jnp.float32),
            pltpu.VMEM((_L, _SPW), jnp.int32),
            pltpu.VMEM((_L, _SPW), jnp.float32),
            pltpu.VMEM((_SPW,), jnp.float32),
            pltpu.VMEM((128,), jnp.float32),
            pltpu.SemaphoreType.DMA,
            pltpu.SemaphoreType.DMA,
        ],
        compiler_params=pltpu.CompilerParams(
            needs_layout_passes=False, use_tc_tiling_on_sc=True),
    )(_sc_body)
    return run(tokT, p, b16)


def kernel(token_ids, embedding_table, W, b):
    tT = embedding_table.T
    tokT = token_ids.T
    w2 = (W * (1.0 / _L)).astype(jnp.float32)
    b16 = jnp.broadcast_to(b, (128,)).astype(jnp.float32)
    p = _matvec(tT, w2)
    out = _sc_gather(tokT, p, b16)
    return out.reshape(_B, 1)


# speedup vs baseline: 1.1165x; 1.0022x over previous
import functools

import jax
import jax.numpy as jnp
from jax import lax
from jax.experimental import pallas as pl
from jax.experimental.pallas import tpu as pltpu
from jax.experimental.pallas import tpu_sc as plsc

_VOCAB = 1000000
_D = 32
_B = 4096
_L = 200
_NC = 2
_NS = 16
_NW = _NC * _NS
_SPW = _B // _NW
_BLKV = 65536
_VPAD = 1001472
_GRID = (_VPAD + _BLKV - 1) // _BLKV


def _mv_body(tT_ref, w_ref, p_ref):
    p_ref[...] = jnp.sum(tT_ref[...] * w_ref[...], axis=0)


def _matvec(tT, w2):
    return pl.pallas_call(
        _mv_body,
        grid=(_GRID,),
        in_specs=[
            pl.BlockSpec((_D, _BLKV), lambda i: (0, i)),
            pl.BlockSpec((_D, 1), lambda i: (0, 0)),
        ],
        out_specs=pl.BlockSpec((_BLKV,), lambda i: (i,)),
        out_shape=jax.ShapeDtypeStruct((_VPAD,), jnp.float32),
        compiler_params=pltpu.CompilerParams(
            dimension_semantics=("arbitrary",),
        ),
    )(tT, w2)


_PSLICE = _VPAD // 16


def _sc_body(tokT_hbm, p_hbm, b_hbm, out_hbm,
             p_sh, idx_v, vals_v, out_v, b_v, sem, psem):
    cid = lax.axis_index("c")
    sid = lax.axis_index("s")
    wid = sid * _NC + cid

    off = sid * _PSLICE
    pcopy = pltpu.async_copy(p_hbm.at[pl.ds(off, _PSLICE)],
                             p_sh.at[pl.ds(off, _PSLICE)], psem)
    pltpu.sync_copy(b_hbm, b_v)
    col = wid * _SPW
    pltpu.sync_copy(tokT_hbm.at[:, pl.ds(col, _SPW)], idx_v)
    pcopy.wait()
    plsc.subcore_barrier()

    bias = b_v[pl.ds(0, 16)][0]
    nch = 4
    cl = _L // nch

    def fire_chunk(c, s):
        def fire(t, cc):
            pltpu.async_copy(p_sh.at[idx_v.at[t]], vals_v.at[t], s)
            return cc
        lax.fori_loop(c * cl, (c + 1) * cl, fire, 0)

    def wait_chunk(s):
        def drain(t, cc):
            pltpu.make_async_copy(
                p_sh.at[idx_v.at[0]], vals_v.at[0], s).wait()
            return cc
        lax.fori_loop(0, cl, drain, 0)

    def accum_chunk(c, acc):
        def tok_body(t, a):
            return tuple(
                a[g] + vals_v[t, pl.ds(16 * g, 16)]
                for g in range(_SPW // 16)
            )
        return lax.fori_loop(c * cl, (c + 1) * cl, tok_body, acc)

    sems = (sem, psem)
    fire_chunk(0, sems[0])
    fire_chunk(1, sems[1])
    acc = (jnp.zeros((16,), jnp.float32),) * (_SPW // 16)
    for c in range(nch):
        wait_chunk(sems[c % 2])
        if c + 2 < nch:
            fire_chunk(c + 2, sems[c % 2])
        acc = accum_chunk(c, acc)
    for g in range(_SPW // 16):
        out_v[pl.ds(16 * g, 16)] = acc[g] + bias
    pltpu.sync_copy(out_v, out_hbm.at[pl.ds(col, _SPW)])


def _sc_gather(tokT, p, b16):
    mesh = plsc.VectorSubcoreMesh(core_axis_name="c", subcore_axis_name="s")
    run = functools.partial(
        pl.kernel,
        mesh=mesh,
        out_type=jax.ShapeDtypeStruct((_B,), jnp.float32),
        scratch_types=[
            pltpu.VMEM_SHARED((_VPAD,), jnp.float32),
            pltpu.VMEM((_L, _SPW), jnp.int32),
            pltpu.VMEM((_L, _SPW), jnp.float32),
            pltpu.VMEM((_SPW,), jnp.float32),
            pltpu.VMEM((128,), jnp.float32),
            pltpu.SemaphoreType.DMA,
            pltpu.SemaphoreType.DMA,
        ],
        compiler_params=pltpu.CompilerParams(
            needs_layout_passes=False, use_tc_tiling_on_sc=True),
    )(_sc_body)
    return run(tokT, p, b16)


def kernel(token_ids, embedding_table, W, b):
    tT = embedding_table.T
    tokT = token_ids.T
    w2 = (W * (1.0 / _L)).astype(jnp.float32)
    b16 = jnp.broadcast_to(b, (128,)).astype(jnp.float32)
    p = _matvec(tT, w2)
    out = _sc_gather(tokT, p, b16)
    return out.reshape(_B, 1)
